# baseline (speedup 1.0000x reference)
import jax
import jax.numpy as jnp
from jax import lax
from jax.experimental import pallas as pl
from jax.experimental.pallas import tpu as pltpu
from jax.experimental.pallas import tpu_sc as plsc

NUM_USERS = 100000
NUM_ITEMS = 100000
EMBED_DIM = 64
BATCH = 16384

_info = plsc.get_sparse_core_info()
_NC, _NS, _L = _info.num_cores, _info.num_subcores, _info.num_lanes
_NW = _NC * _NS
_BPW = BATCH // _NW
_CHUNK = 256
_NCHUNK = _BPW // _CHUNK
_ROWS_PER_BLK = _L
_NBLK = _CHUNK // _ROWS_PER_BLK
_PAIR = 2 * EMBED_DIM


def _sc_body(uids_hbm, iids_hbm, utab_hbm, itab_hbm, out_hbm,
             uidx_v, iidx_v, upair_v, ipair_v, urows_v, irows_v,
             out_v, part_v, sem_u, sem_i):
    wid = lax.axis_index("s") * _NC + lax.axis_index("c")
    base = wid * _BPW

    pltpu.sync_copy(uids_hbm.at[pl.ds(base, _BPW)], uidx_v)
    pltpu.sync_copy(iids_hbm.at[pl.ds(base, _BPW)], iidx_v)

    def halve(i, _):
        upair_v[pl.ds(i * _L, _L)] = uidx_v[pl.ds(i * _L, _L)] >> 1
        ipair_v[pl.ds(i * _L, _L)] = iidx_v[pl.ds(i * _L, _L)] >> 1
        return ()

    lax.fori_loop(0, _BPW // _L, halve, (), unroll=False)

    lane = lax.iota(jnp.int32, _L)
    one = jnp.ones((_L,), jnp.int32)

    for c in range(_NCHUNK):
        cu = pltpu.async_copy(
            utab_hbm.at[upair_v.at[pl.ds(c * _CHUNK, _CHUNK)]], urows_v, sem_u)
        ci = pltpu.async_copy(
            itab_hbm.at[ipair_v.at[pl.ds(c * _CHUNK, _CHUNK)]], irows_v, sem_i)
        cu.wait()
        ci.wait()

        def blk(b, _):
            r0 = b * _ROWS_PER_BLK
            for k in range(_ROWS_PER_BLK):
                ubc = plsc.load_gather(
                    uidx_v, [jnp.full((_L,), c * _CHUNK + r0 + k, jnp.int32)])
                ibc = plsc.load_gather(
                    iidx_v, [jnp.full((_L,), c * _CHUNK + r0 + k, jnp.int32)])
                mu = (ubc & one) == one
                mi = (ibc & one) == one
                acc = None
                for d in range(EMBED_DIM // _L):
                    u_lo = urows_v[r0 + k, pl.ds(d * _L, _L)]
                    u_hi = urows_v[r0 + k, pl.ds(EMBED_DIM + d * _L, _L)]
                    i_lo = irows_v[r0 + k, pl.ds(d * _L, _L)]
                    i_hi = irows_v[r0 + k, pl.ds(EMBED_DIM + d * _L, _L)]
                    prod = jnp.where(mu, u_hi, u_lo) * jnp.where(mi, i_hi, i_lo)
                    acc = prod if acc is None else acc + prod
                part_v[pl.ds(k * _L, _L)] = acc
            rowbase = lane * _L
            total = plsc.load_gather(part_v, [rowbase])
            for j in range(1, _L):
                total = total + plsc.load_gather(part_v, [rowbase + j])
            out_v[pl.ds(c * _CHUNK + r0, _L)] = 1.0 / (1.0 + jnp.exp(-total))
            return ()

        lax.fori_loop(0, _NBLK, blk, (), unroll=False)

    pltpu.sync_copy(out_v, out_hbm.at[pl.ds(base, _BPW)])


@jax.jit
def kernel(user_ids, item_ids, user_table, item_table):
    mesh = plsc.VectorSubcoreMesh(core_axis_name="c", subcore_axis_name="s")
    run = pl.kernel(
        _sc_body,
        out_type=jax.ShapeDtypeStruct((BATCH,), jnp.float32),
        mesh=mesh,
        scratch_types=[
            pltpu.VMEM((_BPW,), jnp.int32),
            pltpu.VMEM((_BPW,), jnp.int32),
            pltpu.VMEM((_BPW,), jnp.int32),
            pltpu.VMEM((_BPW,), jnp.int32),
            pltpu.VMEM((_CHUNK, _PAIR), jnp.float32),
            pltpu.VMEM((_CHUNK, _PAIR), jnp.float32),
            pltpu.VMEM((_BPW,), jnp.float32),
            pltpu.VMEM((_L * _L,), jnp.float32),
            pltpu.SemaphoreType.DMA,
            pltpu.SemaphoreType.DMA,
        ],
        compiler_params=pltpu.CompilerParams(
            needs_layout_passes=False, use_tc_tiling_on_sc=True),
    )
    return run(user_ids.astype(jnp.int32), item_ids.astype(jnp.int32),
               user_table.reshape(NUM_USERS // 2, _PAIR),
               item_table.reshape(NUM_ITEMS // 2, _PAIR))
